# baseline (speedup 1.0000x reference)
import functools

import jax
import jax.numpy as jnp
from jax import lax
from jax.experimental import pallas as pl
from jax.experimental.pallas import tpu as pltpu
from jax.experimental.pallas import tpu_sc as plsc

_K = 20
_NEG = 0.2
_EPS = 1e-6


def _topk_body(xt_ref, idx_ref, *, nb, n, k):
    b = pl.program_id(0)
    i = pl.program_id(1)
    xt = xt_ref[0]
    xc = xt_ref[0, pl.ds(i * nb, nb), :]
    xx = jnp.sum(xt * xt, axis=1)
    xxc = jnp.sum(xc * xc, axis=1)
    inner = jax.lax.dot_general(
        xc, xt, (((1,), (1,)), ((), ())),
        preferred_element_type=jnp.float32)
    pair = 2.0 * inner - xxc[:, None] - xx[None, :]

    iota = jax.lax.broadcasted_iota(jnp.int32, (nb, n), 1)
    rows = jax.lax.broadcasted_iota(jnp.int32, (nb, n), 0) + i * nb
    m = jnp.where(iota == rows, -jnp.inf, pair)
    base = b * n
    cols = [rows[:, :1] + base]
    for _ in range(k - 1):
        mx = jnp.max(m, axis=1, keepdims=True)
        first = jnp.min(jnp.where(m == mx, iota, n), axis=1, keepdims=True)
        m = jnp.where(iota == first, -jnp.inf, m)
        cols.append(first + base)
    idx_ref[0] = jnp.concatenate(cols, axis=1)


def _sc_gather(table, idx):
    E = idx.shape[0]
    D = table.shape[1]
    info = plsc.get_sparse_core_info()
    nw = info.num_cores * info.num_subcores
    b_per_w = E // nw
    chunk = 256
    assert b_per_w % chunk == 0
    mesh = plsc.VectorSubcoreMesh(core_axis_name="c", subcore_axis_name="s")

    @functools.partial(
        pl.kernel, mesh=mesh,
        out_type=jax.ShapeDtypeStruct((E, D), jnp.float32),
        scratch_types=[
            pltpu.VMEM((chunk,), jnp.int32),
            pltpu.VMEM((chunk, D), jnp.float32),
            pltpu.SemaphoreType.DMA,
        ],
    )
    def gk(table_hbm, idx_hbm, out_hbm, idx_v, rows_v, sem):
        wid = lax.axis_index("s") * info.num_cores + lax.axis_index("c")
        base = wid * b_per_w
        for c in range(b_per_w // chunk):
            off = base + c * chunk
            pltpu.sync_copy(idx_hbm.at[pl.ds(off, chunk)], idx_v)
            pltpu.async_copy(table_hbm.at[idx_v], rows_v, sem).wait()
            pltpu.sync_copy(rows_v, out_hbm.at[pl.ds(off, chunk)])

    return gk(table, idx)


def _edge_body(g_ref, xc_ref, wn_ref, wc_ref, out_ref, *, rb, k):
    xc = xc_ref[...]
    cpd = jnp.dot(xc, wc_ref[...], preferred_element_type=jnp.float32)
    cp = cpd[:, :192]
    cd = cpd[:, 192:]
    acc = jnp.zeros((rb, 192), dtype=jnp.float32)
    for t in range(k):
        gpd = jnp.dot(g_ref[t], wn_ref[...], preferred_element_type=jnp.float32)
        p = gpd[:, :192] + cp
        d = gpd[:, 192:] + cd
        p0, p1, p2 = p[:, :64], p[:, 64:128], p[:, 128:]
        d0, d1, d2 = d[:, :64], d[:, 64:128], d[:, 128:]
        dot = p0 * d0 + p1 * d1 + p2 * d2
        dnsq = d0 * d0 + d1 * d1 + d2 * d2
        coef = (1.0 - _NEG) * jnp.minimum(dot, 0.0) / (dnsq + _EPS)
        coef3 = jnp.concatenate([coef, coef, coef], axis=1)
        acc = acc + (p - coef3 * d)
    out_ref[...] = acc * (1.0 / k)


def kernel(x, W_lin, W_dir):
    B, C, V, N = x.shape
    x_t = jnp.transpose(x.reshape(B, C * V, N), (0, 2, 1))
    O = W_lin.shape[0]
    W1 = W_lin[:, :C]
    W2 = W_lin[:, C:]
    D = W_dir @ W_lin
    D1 = D[:, :C]
    D2 = D[:, C:]
    eye = jnp.eye(V, dtype=x.dtype)

    def expand(w):
        return jnp.einsum('oc,vw->cvwo', w, eye).reshape(C * V, V * O)

    Wn = jnp.concatenate([expand(W1), expand(D1)], axis=1)
    Wc = jnp.concatenate([expand(W2 - W1), expand(D2 - D1)], axis=1)

    nb = 512
    idx = pl.pallas_call(
        functools.partial(_topk_body, nb=nb, n=N, k=_K),
        grid=(B, N // nb),
        in_specs=[pl.BlockSpec((1, N, C * V), lambda b, i: (b, 0, 0))],
        out_specs=pl.BlockSpec((1, nb, _K), lambda b, i: (b, i, 0)),
        out_shape=jax.ShapeDtypeStruct((B, N, _K), jnp.int32),
    )(x_t)

    idx_t = jnp.transpose(idx, (2, 0, 1)).reshape(-1)
    dp = 128
    table = jnp.pad(x_t.reshape(B * N, C * V), ((0, 0), (0, dp - C * V)))
    g = _sc_gather(table, idx_t).reshape(_K, B * N, dp)

    Wn = jnp.pad(Wn, ((0, dp - C * V), (0, 0)))
    Wc = jnp.pad(Wc, ((0, dp - C * V), (0, 0)))
    rb = 512
    out = pl.pallas_call(
        functools.partial(_edge_body, rb=rb, k=_K),
        grid=(B * N // rb,),
        in_specs=[
            pl.BlockSpec((_K, rb, dp), lambda e: (0, e, 0)),
            pl.BlockSpec((rb, dp), lambda e: (e, 0)),
            pl.BlockSpec((dp, 2 * V * O), lambda e: (0, 0)),
            pl.BlockSpec((dp, 2 * V * O), lambda e: (0, 0)),
        ],
        out_specs=pl.BlockSpec((rb, V * O), lambda e: (e, 0)),
        out_shape=jax.ShapeDtypeStruct((B * N, V * O), jnp.float32),
    )(g, table, Wn, Wc)
    return out.reshape(B, N, V, O).transpose(0, 3, 2, 1)


# speedup vs baseline: 1.0616x; 1.0616x over previous
import functools

import jax
import jax.numpy as jnp
from jax import lax
from jax.experimental import pallas as pl
from jax.experimental.pallas import tpu as pltpu
from jax.experimental.pallas import tpu_sc as plsc

_K = 20
_NEG = 0.2
_EPS = 1e-6


def _topk_body(xt_ref, idx_ref, *, nb, n, k):
    i = pl.program_id(0)
    xt = xt_ref[...]
    xc = xt_ref[pl.ds(i * nb, nb), :]
    xx = jnp.sum(xt * xt, axis=1)
    xxc = jnp.sum(xc * xc, axis=1)
    inner = jax.lax.dot_general(
        xc, xt, (((1,), (1,)), ((), ())),
        preferred_element_type=jnp.float32)
    pair = 2.0 * inner - xxc[:, None] - xx[None, :]

    iota = jax.lax.broadcasted_iota(jnp.int32, (nb, n), 1)
    rows = jax.lax.broadcasted_iota(jnp.int32, (nb, n), 0) + i * nb
    m = jnp.where(iota == rows, -jnp.inf, pair)
    cols = [rows[:, :1]]
    for _ in range(k - 1):
        mx = jnp.max(m, axis=1, keepdims=True)
        first = jnp.min(jnp.where(m == mx, iota, n), axis=1, keepdims=True)
        m = jnp.where(iota == first, -jnp.inf, m)
        cols.append(first)
    idx_ref[...] = jnp.concatenate(cols, axis=1)


def _sc_gather(table, idx):
    E = idx.shape[0]
    D = table.shape[1]
    info = plsc.get_sparse_core_info()
    nw = info.num_cores * info.num_subcores
    b_per_w = E // nw
    chunk = 128
    nchunks = b_per_w // chunk
    assert b_per_w % chunk == 0
    mesh = plsc.VectorSubcoreMesh(core_axis_name="c", subcore_axis_name="s")

    @functools.partial(
        pl.kernel, mesh=mesh,
        out_type=jax.ShapeDtypeStruct((E, D), table.dtype),
        scratch_types=[
            pltpu.VMEM((2, chunk), jnp.int32),
            pltpu.VMEM((2, chunk, D), table.dtype),
            pltpu.SemaphoreType.DMA,
            pltpu.SemaphoreType.DMA,
        ],
    )
    def gk(table_hbm, idx_hbm, out_hbm, idx_v, rows_v, sem0, sem1):
        wid = lax.axis_index("s") * info.num_cores + lax.axis_index("c")
        base = wid * b_per_w
        sems = (sem0, sem1)
        cps = [None, None]
        pltpu.sync_copy(idx_hbm.at[pl.ds(base, chunk)], idx_v.at[0])
        cps[0] = pltpu.async_copy(table_hbm.at[idx_v.at[0]], rows_v.at[0],
                                  sems[0])
        for c in range(1, nchunks):
            buf = c % 2
            pltpu.sync_copy(idx_hbm.at[pl.ds(base + c * chunk, chunk)],
                            idx_v.at[buf])
            cps[buf] = pltpu.async_copy(table_hbm.at[idx_v.at[buf]],
                                        rows_v.at[buf], sems[buf])
            pbuf = (c - 1) % 2
            cps[pbuf].wait()
            pltpu.sync_copy(rows_v.at[pbuf],
                            out_hbm.at[pl.ds(base + (c - 1) * chunk, chunk)])
        last = (nchunks - 1) % 2
        cps[last].wait()
        pltpu.sync_copy(rows_v.at[last],
                        out_hbm.at[pl.ds(base + (nchunks - 1) * chunk, chunk)])

    return gk(table, idx)


def _edge_body(g_ref, xc_ref, wn_ref, wc_ref, out_ref, *, rb, k):
    xc = xc_ref[...]
    cpd = jnp.dot(xc, wc_ref[...], preferred_element_type=jnp.float32)
    cp = cpd[:, :192]
    cd = cpd[:, 192:]
    acc = jnp.zeros((rb, 192), dtype=jnp.float32)
    for t in range(k):
        gpd = jnp.dot(g_ref[t], wn_ref[...], preferred_element_type=jnp.float32)
        p = gpd[:, :192] + cp
        d = gpd[:, 192:] + cd
        p0, p1, p2 = p[:, :64], p[:, 64:128], p[:, 128:]
        d0, d1, d2 = d[:, :64], d[:, 64:128], d[:, 128:]
        dot = p0 * d0 + p1 * d1 + p2 * d2
        dnsq = d0 * d0 + d1 * d1 + d2 * d2
        coef = (1.0 - _NEG) * jnp.minimum(dot, 0.0) / (dnsq + _EPS)
        coef3 = jnp.concatenate([coef, coef, coef], axis=1)
        acc = acc + (p - coef3 * d)
    out_ref[...] = acc * (1.0 / k)


def kernel(x, W_lin, W_dir):
    B, C, V, N = x.shape
    x_t = jnp.transpose(x.reshape(B, C * V, N), (0, 2, 1))
    O = W_lin.shape[0]
    W1 = W_lin[:, :C]
    W2 = W_lin[:, C:]
    D = W_dir @ W_lin
    D1 = D[:, :C]
    D2 = D[:, C:]
    eye = jnp.eye(V, dtype=x.dtype)

    def expand(w):
        return jnp.einsum('oc,vw->cvwo', w, eye).reshape(C * V, V * O)

    Wn = jnp.concatenate([expand(W1), expand(D1)], axis=1)
    Wc = jnp.concatenate([expand(W2 - W1), expand(D2 - D1)], axis=1)

    dp = 128
    Wn = jnp.pad(Wn, ((0, dp - C * V), (0, 0)))
    Wc = jnp.pad(Wc, ((0, dp - C * V), (0, 0)))

    nb = 512
    rb = 512
    outs = []
    for b in range(B):
        xtb = x_t[b]
        idx = pl.pallas_call(
            functools.partial(_topk_body, nb=nb, n=N, k=_K),
            grid=(N // nb,),
            in_specs=[pl.BlockSpec((N, C * V), lambda i: (0, 0))],
            out_specs=pl.BlockSpec((nb, _K), lambda i: (i, 0)),
            out_shape=jax.ShapeDtypeStruct((N, _K), jnp.int32),
        )(xtb)

        idx_t = jnp.transpose(idx, (1, 0)).reshape(-1)
        tableb = jnp.pad(xtb, ((0, 0), (0, dp - C * V)))
        g = _sc_gather(tableb, idx_t).reshape(_K, N, dp)

        outs.append(pl.pallas_call(
            functools.partial(_edge_body, rb=rb, k=_K),
            grid=(N // rb,),
            in_specs=[
                pl.BlockSpec((_K, rb, dp), lambda e: (0, e, 0)),
                pl.BlockSpec((rb, dp), lambda e: (e, 0)),
                pl.BlockSpec((dp, 2 * V * O), lambda e: (0, 0)),
                pl.BlockSpec((dp, 2 * V * O), lambda e: (0, 0)),
            ],
            out_specs=pl.BlockSpec((rb, V * O), lambda e: (e, 0)),
            out_shape=jax.ShapeDtypeStruct((N, V * O), jnp.float32),
        )(g, tableb, Wn, Wc))
    out = jnp.stack(outs)
    return out.reshape(B, N, V, O).transpose(0, 3, 2, 1)


# speedup vs baseline: 1.3200x; 1.2435x over previous
import functools

import jax
import jax.numpy as jnp
from jax.experimental import pallas as pl
from jax.experimental.pallas import tpu as pltpu

_K = 20
_NEG = 0.2
_EPS = 1e-6


def _body(xt_ref, wn_ref, wc_ref, out_ref, *, nb, n, k):
    i = pl.program_id(1)
    xt = xt_ref[0]
    xc = xt_ref[0, pl.ds(i * nb, nb), :]
    xx = jnp.sum(xt * xt, axis=1)
    xxc = jnp.sum(xc * xc, axis=1)
    inner = jax.lax.dot_general(
        xc, xt, (((1,), (1,)), ((), ())),
        preferred_element_type=jnp.float32)
    pair = 2.0 * inner - xxc[:, None] - xx[None, :]

    cpd = jnp.dot(xc, wc_ref[...], preferred_element_type=jnp.float32)
    cp = cpd[:, :192]
    cd = cpd[:, 192:]
    cnd = jnp.dot(xc, wn_ref[...], preferred_element_type=jnp.float32)
    cps = cnd[:, :192]
    cds = cnd[:, 192:]

    xtb = xt.astype(jnp.bfloat16)
    iota = jax.lax.broadcasted_iota(jnp.int32, (nb, n), 1)
    rows = jax.lax.broadcasted_iota(jnp.int32, (nb, n), 0) + i * nb
    m = jnp.where(iota == rows, -jnp.inf, pair)
    ps = cp + cps
    ds = cd + cds
    s0, s1, s2 = ps[:, :64], ps[:, 64:128], ps[:, 128:]
    t0, t1, t2 = ds[:, :64], ds[:, 64:128], ds[:, 128:]
    sdot = s0 * t0 + s1 * t1 + s2 * t2
    sdn = t0 * t0 + t1 * t1 + t2 * t2
    scoef = (1.0 - _NEG) * jnp.minimum(sdot, 0.0) / (sdn + _EPS)
    acc = ps - jnp.concatenate([scoef, scoef, scoef], axis=1) * ds
    for _ in range(k - 1):
        mx = jnp.max(m, axis=1, keepdims=True)
        first = jnp.min(jnp.where(m == mx, iota, n), axis=1, keepdims=True)
        oh = iota == first
        m = jnp.where(oh, -jnp.inf, m)
        g = jnp.dot(oh.astype(jnp.bfloat16), xtb,
                    preferred_element_type=jnp.float32)
        gpd = jnp.dot(g, wn_ref[...], preferred_element_type=jnp.float32)
        p = gpd[:, :192] + cp
        d = gpd[:, 192:] + cd
        p0, p1, p2 = p[:, :64], p[:, 64:128], p[:, 128:]
        d0, d1, d2 = d[:, :64], d[:, 64:128], d[:, 128:]
        dot = p0 * d0 + p1 * d1 + p2 * d2
        dnsq = d0 * d0 + d1 * d1 + d2 * d2
        coef = (1.0 - _NEG) * jnp.minimum(dot, 0.0) / (dnsq + _EPS)
        coef3 = jnp.concatenate([coef, coef, coef], axis=1)
        acc = acc + (p - coef3 * d)
    out_ref[0] = acc * (1.0 / k)


def kernel(x, W_lin, W_dir):
    B, C, V, N = x.shape
    x_t = jnp.transpose(x.reshape(B, C * V, N), (0, 2, 1))
    O = W_lin.shape[0]
    W1 = W_lin[:, :C]
    W2 = W_lin[:, C:]
    D = W_dir @ W_lin
    D1 = D[:, :C]
    D2 = D[:, C:]
    eye = jnp.eye(V, dtype=x.dtype)

    def expand(w):
        return jnp.einsum('oc,vw->cvwo', w, eye).reshape(C * V, V * O)

    Wn = jnp.concatenate([expand(W1), expand(D1)], axis=1)
    Wc = jnp.concatenate([expand(W2 - W1), expand(D2 - D1)], axis=1)

    nb = 512
    body = functools.partial(_body, nb=nb, n=N, k=_K)
    out = pl.pallas_call(
        body,
        grid=(B, N // nb),
        in_specs=[
            pl.BlockSpec((1, N, C * V), lambda b, i: (b, 0, 0)),
            pl.BlockSpec((C * V, 2 * V * O), lambda b, i: (0, 0)),
            pl.BlockSpec((C * V, 2 * V * O), lambda b, i: (0, 0)),
        ],
        out_specs=pl.BlockSpec((1, nb, V * O), lambda b, i: (b, i, 0)),
        out_shape=jax.ShapeDtypeStruct((B, N, V * O), jnp.float32),
    )(x_t, Wn, Wc)
    return out.reshape(B, N, V, O).transpose(0, 3, 2, 1)


# speedup vs baseline: 1.3553x; 1.0268x over previous
import functools

import jax
import jax.numpy as jnp
from jax.experimental import pallas as pl
from jax.experimental.pallas import tpu as pltpu

_K = 20
_NEG = 0.2
_EPS = 1e-6


def _body(xt_ref, wn_ref, wc_ref, out_ref, *, nb, n, k):
    i = pl.program_id(1)
    xt = xt_ref[0]
    xc = xt_ref[0, pl.ds(i * nb, nb), :]
    xx = jnp.sum(xt * xt, axis=1)
    xxc = jnp.sum(xc * xc, axis=1)
    inner = jax.lax.dot_general(
        xc, xt, (((1,), (1,)), ((), ())),
        preferred_element_type=jnp.float32)
    pair = 2.0 * inner - xxc[:, None] - xx[None, :]

    cpd = jnp.dot(xc, wc_ref[...], preferred_element_type=jnp.float32)
    cp = cpd[:, :192]
    cd = cpd[:, 192:]
    cnd = jnp.dot(xc, wn_ref[...], preferred_element_type=jnp.float32)
    cps = cnd[:, :192]
    cds = cnd[:, 192:]

    iota = jax.lax.broadcasted_iota(jnp.int32, (nb, n), 1)
    rows = jax.lax.broadcasted_iota(jnp.int32, (nb, n), 0) + i * nb
    m = jnp.where(iota == rows, -jnp.inf, pair)
    ps = cp + cps
    ds = cd + cds
    s0, s1, s2 = ps[:, :64], ps[:, 64:128], ps[:, 128:]
    t0, t1, t2 = ds[:, :64], ds[:, 64:128], ds[:, 128:]
    sdot = s0 * t0 + s1 * t1 + s2 * t2
    sdn = t0 * t0 + t1 * t1 + t2 * t2
    scoef = (1.0 - _NEG) * jnp.minimum(sdot, 0.0) / (sdn + _EPS)
    acc = ps - jnp.concatenate([scoef, scoef, scoef], axis=1) * ds
    for _ in range(k - 1):
        mx = jnp.max(m, axis=1, keepdims=True)
        first = jnp.min(jnp.where(m == mx, iota, n), axis=1, keepdims=True)
        oh = iota == first
        m = jnp.where(oh, -jnp.inf, m)
        g = jnp.dot(oh.astype(jnp.float32), xt,
                    preferred_element_type=jnp.float32)
        gpd = jnp.dot(g, wn_ref[...], preferred_element_type=jnp.float32)
        p = gpd[:, :192] + cp
        d = gpd[:, 192:] + cd
        p0, p1, p2 = p[:, :64], p[:, 64:128], p[:, 128:]
        d0, d1, d2 = d[:, :64], d[:, 64:128], d[:, 128:]
        dot = p0 * d0 + p1 * d1 + p2 * d2
        dnsq = d0 * d0 + d1 * d1 + d2 * d2
        coef = (1.0 - _NEG) * jnp.minimum(dot, 0.0) / (dnsq + _EPS)
        coef3 = jnp.concatenate([coef, coef, coef], axis=1)
        acc = acc + (p - coef3 * d)
    out_ref[0] = acc * (1.0 / k)


def kernel(x, W_lin, W_dir):
    B, C, V, N = x.shape
    x_t = jnp.transpose(x.reshape(B, C * V, N), (0, 2, 1))
    O = W_lin.shape[0]
    W1 = W_lin[:, :C]
    W2 = W_lin[:, C:]
    D = W_dir @ W_lin
    D1 = D[:, :C]
    D2 = D[:, C:]
    eye = jnp.eye(V, dtype=x.dtype)

    def expand(w):
        return jnp.einsum('oc,vw->cvwo', w, eye).reshape(C * V, V * O)

    Wn = jnp.concatenate([expand(W1), expand(D1)], axis=1)
    Wc = jnp.concatenate([expand(W2 - W1), expand(D2 - D1)], axis=1)

    nb = 512
    body = functools.partial(_body, nb=nb, n=N, k=_K)
    out = pl.pallas_call(
        body,
        grid=(B, N // nb),
        in_specs=[
            pl.BlockSpec((1, N, C * V), lambda b, i: (b, 0, 0)),
            pl.BlockSpec((C * V, 2 * V * O), lambda b, i: (0, 0)),
            pl.BlockSpec((C * V, 2 * V * O), lambda b, i: (0, 0)),
        ],
        out_specs=pl.BlockSpec((1, nb, V * O), lambda b, i: (b, i, 0)),
        out_shape=jax.ShapeDtypeStruct((B, N, V * O), jnp.float32),
        compiler_params=pltpu.CompilerParams(
            dimension_semantics=("parallel", "parallel")),
    )(x_t, Wn, Wc)
    return out.reshape(B, N, V, O).transpose(0, 3, 2, 1)
